# Initial kernel scaffold; baseline (speedup 1.0000x reference)
import jax
import jax.numpy as jnp
from jax.experimental import pallas as pl


def kernel(x, adj_indices, adj_values, embedding):
    raise NotImplementedError("write your pallas kernel here")


# speedup vs baseline: 2.6782x; 2.6782x over previous
import functools

import jax
import jax.numpy as jnp
from jax import lax
from jax.experimental import pallas as pl
from jax.experimental.pallas import tpu as pltpu
from jax.experimental.pallas import tpu_sc as plsc

_N = 10000
_D = 128
_NC = 2
_NS = 16
_NW = _NC * _NS
_EBLK = 128
_NP = 10112
_RPT = _NP // _NS
_LP = 64


def _edge_layer_body(e_hbm, cols_hbm, rows_hbm, vals_hbm, out_hbm,
                     acc, colv, rowv, valv, msg, gsem, ssem, *, nblk):
    c = lax.axis_index("c")
    s = lax.axis_index("s")
    wid = s * _NC + c

    z16 = jnp.zeros((16,), jnp.float32)

    def _zrow(i, carry):
        for h in range(8):
            msg[i, pl.ds(16 * h, 16)] = z16
        return carry

    lax.fori_loop(0, _EBLK, _zrow, 0)
    for off, sz in ((0, 128), (128, 128), (256, 128), (384, 128), (512, 120)):
        pltpu.sync_copy(msg.at[pl.ds(0, sz)],
                        acc.at[pl.ds(s * _RPT + off, sz)])
    plsc.subcore_barrier()

    pltpu.sync_copy(cols_hbm.at[wid], colv)
    pltpu.sync_copy(rows_hbm.at[wid], rowv)
    pltpu.sync_copy(vals_hbm.at[wid], valv)

    def _blk(b, carry):
        pltpu.async_copy(e_hbm.at[colv.at[b]], msg, gsem).wait()

        def _edge(j, fv):
            v = plsc.load_gather(valv, [fv])
            for h in range(8):
                sl = pl.ds(16 * h, 16)
                msg[j, sl] = msg[j, sl] * v
            return fv + 1

        lax.fori_loop(0, _EBLK, _edge,
                      jnp.full((16,), b * _EBLK, jnp.int32), unroll=2)
        pltpu.async_copy(msg, acc.at[rowv.at[b]], ssem, add=True).wait()
        return carry

    lax.fori_loop(0, nblk, _blk, 0)
    plsc.subcore_barrier()
    pltpu.sync_copy(acc.at[pl.ds(s * _RPT, _RPT)],
                    out_hbm.at[c, pl.ds(s * _RPT, _RPT)])


def _edge_layer(e, cols_p, rows_p, vals_p, nblk):
    mesh = plsc.VectorSubcoreMesh(core_axis_name="c", subcore_axis_name="s")
    f = pl.kernel(
        functools.partial(_edge_layer_body, nblk=nblk),
        out_type=jax.ShapeDtypeStruct((_NC, _NP, _D), jnp.float32),
        mesh=mesh,
        compiler_params=pltpu.CompilerParams(needs_layout_passes=False),
        scratch_types=[
            pltpu.VMEM_SHARED((_NP, _D), jnp.float32),
            pltpu.VMEM((nblk, _EBLK), jnp.int32),
            pltpu.VMEM((nblk, _EBLK), jnp.int32),
            pltpu.VMEM((nblk * _EBLK,), jnp.float32),
            pltpu.VMEM((_EBLK, _D), jnp.float32),
            pltpu.SemaphoreType.DMA,
            pltpu.SemaphoreType.DMA,
        ],
    )
    return f(e, cols_p, rows_p, vals_p)


def _relu_combine(p):
    def body(p_ref, o_ref):
        o_ref[...] = jnp.maximum(p_ref[0] + p_ref[1], 0.0)

    return pl.pallas_call(
        body,
        out_shape=jax.ShapeDtypeStruct((_NP, _D), jnp.float32),
        grid=(8,),
        in_specs=[pl.BlockSpec((2, _NP // 8, _D), lambda i: (0, i, 0))],
        out_specs=pl.BlockSpec((_NP // 8, _D), lambda i: (i, 0)),
    )(p)


def _final_mean(p2, e0, e1):
    def body(p_ref, e0_ref, e1_ref, o_ref):
        e2 = jnp.maximum(p_ref[0] + p_ref[1], 0.0)
        o_ref[...] = (e0_ref[...] + e1_ref[...] + e2) * (1.0 / 3.0)

    return pl.pallas_call(
        body,
        out_shape=jax.ShapeDtypeStruct((_NP, _D), jnp.float32),
        grid=(8,),
        in_specs=[
            pl.BlockSpec((2, _NP // 8, _D), lambda i: (0, i, 0)),
            pl.BlockSpec((_NP // 8, _D), lambda i: (i, 0)),
            pl.BlockSpec((_NP // 8, _D), lambda i: (i, 0)),
        ],
        out_specs=pl.BlockSpec((_NP // 8, _D), lambda i: (i, 0)),
    )(p2, e0, e1)


def _patient_body(em_hbm, x_hbm, out_hbm, xv, rowsb, outv, sem, *, ppw):
    c = lax.axis_index("c")
    s = lax.axis_index("s")
    wid = s * _NC + c
    base = wid * ppw
    pltpu.sync_copy(x_hbm.at[pl.ds(base, ppw)], xv)

    def _pat(p, carry):
        pltpu.async_copy(em_hbm.at[xv.at[p]], rowsb, sem).wait()
        cnt = jnp.zeros((16,), jnp.int32)
        for k in range(4):
            xs = xv[p, pl.ds(16 * k, 16)]
            cnt = cnt + plsc.all_reduce_population_count(xs != 0)
        inv = 1.0 / jnp.maximum(cnt.astype(jnp.float32), 1.0)
        for h in range(8):
            sl = pl.ds(16 * h, 16)

            def _rs(r, a):
                return a + rowsb[r, sl]

            acc = lax.fori_loop(0, _LP, _rs, jnp.zeros((16,), jnp.float32),
                                unroll=4)
            outv[p, sl] = acc * inv
        return carry

    lax.fori_loop(0, ppw, _pat, 0)
    pltpu.sync_copy(outv, out_hbm.at[pl.ds(base, ppw)])


def _patient_readout(emf, xp, bpat):
    ppw = bpat // _NW
    mesh = plsc.VectorSubcoreMesh(core_axis_name="c", subcore_axis_name="s")
    f = pl.kernel(
        functools.partial(_patient_body, ppw=ppw),
        out_type=jax.ShapeDtypeStruct((bpat, _D), jnp.float32),
        mesh=mesh,
        compiler_params=pltpu.CompilerParams(needs_layout_passes=False),
        scratch_types=[
            pltpu.VMEM((ppw, _LP), jnp.int32),
            pltpu.VMEM((_LP, _D), jnp.float32),
            pltpu.VMEM((ppw, _D), jnp.float32),
            pltpu.SemaphoreType.DMA,
        ],
    )
    return f(emf, xp)


def kernel(x, adj_indices, adj_values, embedding):
    nnz = adj_values.shape[0]
    nblk = -(-nnz // (_NW * _EBLK))
    pad = _NW * nblk * _EBLK - nnz

    e0 = jnp.pad(embedding[1:], ((0, _NP - _N), (0, 0)))
    rows = jnp.concatenate(
        [adj_indices[0], jnp.zeros((pad,), jnp.int32)]).reshape(_NW, nblk, _EBLK)
    cols = jnp.concatenate(
        [adj_indices[1], jnp.zeros((pad,), jnp.int32)]).reshape(_NW, nblk, _EBLK)
    vals = jnp.concatenate(
        [adj_values, jnp.zeros((pad,), jnp.float32)]).reshape(_NW, nblk * _EBLK)

    p1 = _edge_layer(e0, cols, rows, vals, nblk)
    e1 = _relu_combine(p1)
    p2 = _edge_layer(e1, cols, rows, vals, nblk)
    emean = _final_mean(p2, e0, e1)
    emf = jnp.concatenate([jnp.zeros((1, _D), jnp.float32), emean], axis=0)

    bpat, lcur = x.shape
    xp = jnp.pad(x, ((0, 0), (0, _LP - lcur)))
    return _patient_readout(emf, xp, bpat)
